# baseline (speedup 1.0000x reference)
import functools

import jax
import jax.numpy as jnp
from jax import lax
from jax.experimental import pallas as pl
from jax.experimental.pallas import tpu as pltpu
from jax.experimental.pallas import tpu_sc as plsc

BATCH = 4096
SEQ = 200
DIM = 64
VOCAB = 1000000
N = BATCH * SEQ
LANES = 16
GRP = DIM // LANES

P2 = 524288
TW = 4096
TGRID = P2 // TW
NBOFF = P2 // TW

NC = 2
NS = 16
NW = NC * NS
BBLK = BATCH // NW
DT = DIM // 8

_mesh = plsc.VectorSubcoreMesh(
    core_axis_name="c", subcore_axis_name="s", num_cores=NC, num_subcores=NS
)


def _detile_body(in_a, in_b, out_ref):
    out_ref[:, 0:DIM] = in_a[...].T
    out_ref[:, DIM:128] = in_b[...].T


_detile_tc = pl.pallas_call(
    _detile_body,
    grid=(TGRID,),
    in_specs=[
        pl.BlockSpec((DIM, TW), lambda i: (0, i)),
        pl.BlockSpec((DIM, TW), lambda i: (0, jnp.minimum(i + NBOFF, VOCAB // TW))),
    ],
    out_specs=pl.BlockSpec((TW, 128), lambda i: (i, 0)),
    out_shape=jax.ShapeDtypeStruct((P2, 128), jnp.float32),
)


@functools.partial(
    pl.kernel,
    out_type=jax.ShapeDtypeStruct((SEQ, DT, NW, 8, 128), jnp.float32),
    mesh=_mesh,
    compiler_params=pltpu.CompilerParams(
        use_tc_tiling_on_sc=False, needs_layout_passes=False
    ),
    scratch_types=[
        pltpu.VMEM((SEQ, BBLK), jnp.int32),
        [pltpu.VMEM((BBLK, DIM), jnp.float32) for _ in range(4)],
        [pltpu.VMEM((DIM, 129), jnp.float32) for _ in range(4)],
        pltpu.VMEM((SEQ, DIM), jnp.float32),
        [pltpu.SemaphoreType.DMA for _ in range(4)],
        [pltpu.SemaphoreType.DMA for _ in range(4)],
    ],
)
def _emb_sc(idx_hbm, tab_hbm, pos_hbm, out_hbm,
            idx_v, rows, tt, pos_v, sem_g, sem_s):
    wid = lax.axis_index("s") * NC + lax.axis_index("c")

    pltpu.sync_copy(pos_hbm, pos_v)
    pltpu.sync_copy(idx_hbm.at[wid], idx_v)

    dvec = [jnp.arange(LANES, dtype=jnp.int32) + c * LANES for c in range(GRP)]

    def fire_gather(s, rows_ref, sem):
        pltpu.async_copy(tab_hbm.at[idx_v.at[s]], rows_ref, sem)

    def drain_gather(rows_ref, sem):
        pltpu.make_async_copy(tab_hbm.at[pl.ds(0, BBLK)], rows_ref, sem).wait()

    def fire_store(s, t_ref, sem):
        for dt in range(DT):
            pltpu.async_copy(
                t_ref.at[pl.ds(dt * 8, 8), pl.ds(0, 128)],
                out_hbm.at[s, dt, wid].at[pl.ds(0, 8), :],
                sem,
            )

    def drain_store(q, sem):
        pltpu.make_async_copy(tab_hbm.at[pl.ds(0, BBLK)], rows[q], sem).wait()

    def transpose_add(s, rows_ref, t_ref):
        pv = [pos_v[s, pl.ds(c * LANES, LANES)] for c in range(GRP)]

        ones = jnp.full((LANES,), 1, dtype=jnp.int32)

        def body(i, blv):
            bv = blv
            for u in range(4):
                bl = 4 * i + u
                for c in range(GRP):
                    vals = rows_ref[bl, pl.ds(c * LANES, LANES)] + pv[c]
                    plsc.store_scatter(t_ref, [dvec[c], bv], vals)
                bv = bv + ones
            return bv

        lax.fori_loop(0, BBLK // 4, body, jnp.full((LANES,), 0, dtype=jnp.int32))

    for q in range(3):
        fire_gather(q, rows[q], sem_g[q])

    def quad_body(t, _):
        for q in range(4):
            s = 4 * t + q
            nxt = (q + 3) % 4
            if q == 0:
                fire_gather(s + 3, rows[nxt], sem_g[nxt])
            else:
                @pl.when(t < SEQ // 4 - 1)
                def _():
                    fire_gather(s + 3, rows[nxt], sem_g[nxt])

            drain_gather(rows[q], sem_g[q])

            @pl.when(t > 0)
            def _():
                drain_store(q, sem_s[q])

            transpose_add(s, rows[q], tt[q])
            fire_store(s, tt[q], sem_s[q])
        return 0

    lax.fori_loop(0, SEQ // 4, quad_body, 0)
    for q in range(4):
        drain_store(q, sem_s[q])


def kernel(words, word_table, pos_table):
    wt_t = word_table.T
    tab128 = _detile_tc(wt_t, wt_t)
    tab_lin = tab128.reshape(2 * P2, DIM)

    w = words.astype(jnp.int32)
    g = ((w & (P2 - 1)) << 1) | (w >> 19)
    idx3 = g.T.reshape(SEQ, NW, BBLK).transpose(1, 0, 2)

    out5 = _emb_sc(idx3, tab_lin, pos_table)
    return out5.transpose(2, 4, 0, 1, 3).reshape(BATCH, SEQ, DIM)


# speedup vs baseline: 1.7560x; 1.7560x over previous
import functools

import jax
import jax.numpy as jnp
from jax import lax
from jax.experimental import pallas as pl
from jax.experimental.pallas import tpu as pltpu
from jax.experimental.pallas import tpu_sc as plsc

BATCH = 4096
SEQ = 200
DIM = 64
VOCAB = 1000000
N = BATCH * SEQ
LANES = 16
GRP = DIM // LANES

P2 = 524288
TW = 4096
TGRID = P2 // TW
NBOFF = P2 // TW

NC = 2
NS = 16
NW = NC * NS
BBLK = BATCH // NW
DT = DIM // 8

_mesh = plsc.VectorSubcoreMesh(
    core_axis_name="c", subcore_axis_name="s", num_cores=NC, num_subcores=NS
)


def _detile_body(in_a, in_b, out_ref):
    out_ref[:, 0:DIM] = in_a[...].T
    out_ref[:, DIM:128] = in_b[...].T


_detile_tc = pl.pallas_call(
    _detile_body,
    grid=(TGRID,),
    in_specs=[
        pl.BlockSpec((DIM, TW), lambda i: (0, i)),
        pl.BlockSpec((DIM, TW), lambda i: (0, jnp.minimum(i + NBOFF, VOCAB // TW))),
    ],
    out_specs=pl.BlockSpec((TW, 128), lambda i: (i, 0)),
    out_shape=jax.ShapeDtypeStruct((P2, 128), jnp.float32),
)


@functools.partial(
    pl.kernel,
    out_type=jax.ShapeDtypeStruct((SEQ, DT, NW, 8, 128), jnp.float32),
    mesh=_mesh,
    compiler_params=pltpu.CompilerParams(
        use_tc_tiling_on_sc=False, needs_layout_passes=False
    ),
    scratch_types=[
        pltpu.VMEM((SEQ, BBLK), jnp.int32),
        [pltpu.VMEM((BBLK, DIM), jnp.float32) for _ in range(4)],
        [pltpu.VMEM((DT, 8, 129), jnp.float32) for _ in range(4)],
        pltpu.VMEM((SEQ, DIM), jnp.float32),
        [pltpu.SemaphoreType.DMA for _ in range(4)],
        [pltpu.SemaphoreType.DMA for _ in range(4)],
    ],
)
def _emb_sc(idx_hbm, tab_hbm, pos_hbm, out_hbm,
            idx_v, rows, tt, pos_v, sem_g, sem_s):
    wid = lax.axis_index("s") * NC + lax.axis_index("c")

    pltpu.sync_copy(pos_hbm, pos_v)
    pltpu.sync_copy(idx_hbm.at[wid], idx_v)

    dd = [jnp.arange(LANES, dtype=jnp.int32) + c * LANES for c in range(GRP)]
    dtv = [d >> 3 for d in dd]
    drv = [d & 7 for d in dd]

    def fire_gather(s, rows_ref, sem):
        pltpu.async_copy(tab_hbm.at[idx_v.at[s]], rows_ref, sem)

    def drain_gather(rows_ref, sem):
        pltpu.make_async_copy(tab_hbm.at[pl.ds(0, BBLK)], rows_ref, sem).wait()

    def fire_store(s, t_ref, sem):
        pltpu.async_copy(
            t_ref.at[:, :, pl.ds(0, 128)],
            out_hbm.at[s].at[pl.ds(0, DT), wid],
            sem,
        )

    def drain_store(q, sem):
        pltpu.make_async_copy(tab_hbm.at[pl.ds(0, BBLK)], rows[q], sem).wait()

    def transpose_add(s, rows_ref, t_ref):
        pv = [pos_v[s, pl.ds(c * LANES, LANES)] for c in range(GRP)]

        @plsc.parallel_loop(0, BBLK, step=1, unroll=8)
        def _(bl):
            blv = jnp.broadcast_to(bl, (LANES,)).astype(jnp.int32)
            for c in range(GRP):
                vals = rows_ref[bl, pl.ds(c * LANES, LANES)] + pv[c]
                plsc.store_scatter(t_ref, [dtv[c], drv[c], blv], vals)

    for q in range(3):
        fire_gather(q, rows[q], sem_g[q])

    def quad_body(t, _):
        for q in range(4):
            s = 4 * t + q
            nxt = (q + 3) % 4
            if q == 0:
                fire_gather(s + 3, rows[nxt], sem_g[nxt])
            else:
                @pl.when(t < SEQ // 4 - 1)
                def _():
                    fire_gather(s + 3, rows[nxt], sem_g[nxt])

            drain_gather(rows[q], sem_g[q])

            @pl.when(t > 0)
            def _():
                drain_store(q, sem_s[q])

            transpose_add(s, rows[q], tt[q])
            fire_store(s, tt[q], sem_s[q])
        return 0

    lax.fori_loop(0, SEQ // 4, quad_body, 0)
    for q in range(4):
        drain_store(q, sem_s[q])


def kernel(words, word_table, pos_table):
    wt_t = word_table.T
    tab128 = _detile_tc(wt_t, wt_t)
    tab_lin = tab128.reshape(2 * P2, DIM)

    w = words.astype(jnp.int32)
    g = ((w & (P2 - 1)) << 1) | (w >> 19)
    idx3 = g.T.reshape(SEQ, NW, BBLK).transpose(1, 0, 2)

    out5 = _emb_sc(idx3, tab_lin, pos_table)
    return out5.transpose(2, 4, 0, 1, 3).reshape(BATCH, SEQ, DIM)


# speedup vs baseline: 1.9033x; 1.0839x over previous
import functools

import jax
import jax.numpy as jnp
from jax import lax
from jax.experimental import pallas as pl
from jax.experimental.pallas import tpu as pltpu
from jax.experimental.pallas import tpu_sc as plsc

BATCH = 4096
SEQ = 200
DIM = 64
VOCAB = 1000000
N = BATCH * SEQ
LANES = 16
GRP = DIM // LANES

P2 = 524288
TW = 8192
TGRID = P2 // TW
NBOFF = P2 // TW

NC = 2
NS = 16
NW = NC * NS
BBLK = BATCH // NW
DT = DIM // 8

_mesh = plsc.VectorSubcoreMesh(
    core_axis_name="c", subcore_axis_name="s", num_cores=NC, num_subcores=NS
)


def _detile_body(in_a, in_b, out_ref):
    out_ref[:, 0:DIM] = in_a[...].T
    out_ref[:, DIM:128] = in_b[...].T


_detile_tc = pl.pallas_call(
    _detile_body,
    grid=(TGRID,),
    in_specs=[
        pl.BlockSpec((DIM, TW), lambda i: (0, i)),
        pl.BlockSpec((DIM, TW), lambda i: (0, jnp.minimum(i + NBOFF, VOCAB // TW))),
    ],
    out_specs=pl.BlockSpec((TW, 128), lambda i: (i, 0)),
    out_shape=jax.ShapeDtypeStruct((P2, 128), jnp.float32),
)


@functools.partial(
    pl.kernel,
    out_type=jax.ShapeDtypeStruct((SEQ, DT, NW, 8, 128), jnp.float32),
    mesh=_mesh,
    compiler_params=pltpu.CompilerParams(
        use_tc_tiling_on_sc=False, needs_layout_passes=False
    ),
    scratch_types=[
        pltpu.VMEM((SEQ, BBLK), jnp.int32),
        [pltpu.VMEM((BBLK, DIM), jnp.float32) for _ in range(4)],
        [pltpu.VMEM((DT, 8, 129), jnp.float32) for _ in range(4)],
        pltpu.VMEM((SEQ, DIM), jnp.float32),
        [pltpu.SemaphoreType.DMA for _ in range(4)],
        [pltpu.SemaphoreType.DMA for _ in range(4)],
    ],
)
def _emb_sc(idx_hbm, tab_hbm, pos_hbm, out_hbm,
            idx_v, rows, tt, pos_v, sem_g, sem_s):
    wid = lax.axis_index("s") * NC + lax.axis_index("c")

    pltpu.sync_copy(pos_hbm, pos_v)
    pltpu.sync_copy(idx_hbm.at[wid], idx_v)

    dd = [jnp.arange(LANES, dtype=jnp.int32) + c * LANES for c in range(GRP)]
    dtv = [d >> 3 for d in dd]
    drv = [d & 7 for d in dd]

    def fire_gather(s, rows_ref, sem):
        pltpu.async_copy(tab_hbm.at[idx_v.at[s]], rows_ref, sem)

    def drain_gather(rows_ref, sem):
        pltpu.make_async_copy(tab_hbm.at[pl.ds(0, BBLK)], rows_ref, sem).wait()

    def fire_store(s, t_ref, sem):
        pltpu.async_copy(
            t_ref.at[:, :, pl.ds(0, 128)],
            out_hbm.at[s].at[pl.ds(0, DT), wid],
            sem,
        )

    def drain_store(q, sem):
        pltpu.make_async_copy(tab_hbm.at[pl.ds(0, BBLK)], rows[q], sem).wait()

    def transpose_add(s, rows_ref, t_ref):
        pv = [pos_v[s, pl.ds(c * LANES, LANES)] for c in range(GRP)]

        @plsc.parallel_loop(0, BBLK, step=1, unroll=8)
        def _(bl):
            blv = jnp.broadcast_to(bl, (LANES,)).astype(jnp.int32)
            for c in range(GRP):
                vals = rows_ref[bl, pl.ds(c * LANES, LANES)] + pv[c]
                plsc.store_scatter(t_ref, [dtv[c], drv[c], blv], vals)

    for q in range(3):
        fire_gather(q, rows[q], sem_g[q])

    def quad_body(t, _):
        for q in range(4):
            s = 4 * t + q
            nxt = (q + 3) % 4
            if q == 0:
                fire_gather(s + 3, rows[nxt], sem_g[nxt])
            else:
                @pl.when(t < SEQ // 4 - 1)
                def _():
                    fire_gather(s + 3, rows[nxt], sem_g[nxt])

            drain_gather(rows[q], sem_g[q])

            @pl.when(t > 0)
            def _():
                drain_store(q, sem_s[q])

            transpose_add(s, rows[q], tt[q])
            fire_store(s, tt[q], sem_s[q])
        return 0

    lax.fori_loop(0, SEQ // 4, quad_body, 0)
    for q in range(4):
        drain_store(q, sem_s[q])


def kernel(words, word_table, pos_table):
    wt_t = word_table.T
    tab128 = _detile_tc(wt_t, wt_t)
    tab_lin = tab128.reshape(2 * P2, DIM)

    w = words.astype(jnp.int32)
    g = ((w & (P2 - 1)) << 1) | (w >> 19)
    idx3 = g.T.reshape(SEQ, NW, BBLK).transpose(1, 0, 2)

    out5 = _emb_sc(idx3, tab_lin, pos_table)
    return out5.transpose(2, 4, 0, 1, 3).reshape(BATCH, SEQ, DIM)


# speedup vs baseline: 1.9748x; 1.0375x over previous
import functools

import jax
import jax.numpy as jnp
from jax import lax
from jax.experimental import pallas as pl
from jax.experimental.pallas import tpu as pltpu
from jax.experimental.pallas import tpu_sc as plsc

BATCH = 4096
SEQ = 200
DIM = 64
VOCAB = 1000000
N = BATCH * SEQ
LANES = 16
GRP = DIM // LANES

P2 = 524288
TW = 16384
TGRID = P2 // TW
NBOFF = P2 // TW

NC = 2
NS = 16
NW = NC * NS
BBLK = BATCH // NW
DT = DIM // 8

_mesh = plsc.VectorSubcoreMesh(
    core_axis_name="c", subcore_axis_name="s", num_cores=NC, num_subcores=NS
)


def _detile_body(in_a, in_b, out_ref):
    out_ref[:, 0:DIM] = in_a[...].T
    out_ref[:, DIM:128] = in_b[...].T


_detile_tc = pl.pallas_call(
    _detile_body,
    grid=(TGRID,),
    in_specs=[
        pl.BlockSpec((DIM, TW), lambda i: (0, i)),
        pl.BlockSpec((DIM, TW), lambda i: (0, jnp.minimum(i + NBOFF, VOCAB // TW))),
    ],
    out_specs=pl.BlockSpec((TW, 128), lambda i: (i, 0)),
    out_shape=jax.ShapeDtypeStruct((P2, 128), jnp.float32),
    compiler_params=pltpu.CompilerParams(vmem_limit_bytes=100 * 2**20),
)


@functools.partial(
    pl.kernel,
    out_type=jax.ShapeDtypeStruct((SEQ, DT, NW, 8, 128), jnp.float32),
    mesh=_mesh,
    compiler_params=pltpu.CompilerParams(
        use_tc_tiling_on_sc=False, needs_layout_passes=False
    ),
    scratch_types=[
        pltpu.VMEM((SEQ, BBLK), jnp.int32),
        [pltpu.VMEM((BBLK, DIM), jnp.float32) for _ in range(4)],
        [pltpu.VMEM((DT, 8, 129), jnp.float32) for _ in range(4)],
        pltpu.VMEM((SEQ, DIM), jnp.float32),
        [pltpu.SemaphoreType.DMA for _ in range(4)],
        [pltpu.SemaphoreType.DMA for _ in range(4)],
    ],
)
def _emb_sc(idx_hbm, tab_hbm, pos_hbm, out_hbm,
            idx_v, rows, tt, pos_v, sem_g, sem_s):
    wid = lax.axis_index("s") * NC + lax.axis_index("c")

    pltpu.sync_copy(pos_hbm, pos_v)
    pltpu.sync_copy(idx_hbm.at[wid], idx_v)

    dd = [jnp.arange(LANES, dtype=jnp.int32) + c * LANES for c in range(GRP)]
    dtv = [d >> 3 for d in dd]
    drv = [d & 7 for d in dd]

    def fire_gather(s, rows_ref, sem):
        pltpu.async_copy(tab_hbm.at[idx_v.at[s]], rows_ref, sem)

    def drain_gather(rows_ref, sem):
        pltpu.make_async_copy(tab_hbm.at[pl.ds(0, BBLK)], rows_ref, sem).wait()

    def fire_store(s, t_ref, sem):
        pltpu.async_copy(
            t_ref.at[:, :, pl.ds(0, 128)],
            out_hbm.at[s].at[pl.ds(0, DT), wid],
            sem,
        )

    def drain_store(q, sem):
        pltpu.make_async_copy(tab_hbm.at[pl.ds(0, BBLK)], rows[q], sem).wait()

    def transpose_add(s, rows_ref, t_ref):
        pv = [pos_v[s, pl.ds(c * LANES, LANES)] for c in range(GRP)]

        @plsc.parallel_loop(0, BBLK, step=1, unroll=8)
        def _(bl):
            blv = jnp.broadcast_to(bl, (LANES,)).astype(jnp.int32)
            for c in range(GRP):
                vals = rows_ref[bl, pl.ds(c * LANES, LANES)] + pv[c]
                plsc.store_scatter(t_ref, [dtv[c], drv[c], blv], vals)

    for q in range(3):
        fire_gather(q, rows[q], sem_g[q])

    def quad_body(t, _):
        for q in range(4):
            s = 4 * t + q
            nxt = (q + 3) % 4
            if q == 0:
                fire_gather(s + 3, rows[nxt], sem_g[nxt])
            else:
                @pl.when(t < SEQ // 4 - 1)
                def _():
                    fire_gather(s + 3, rows[nxt], sem_g[nxt])

            drain_gather(rows[q], sem_g[q])

            @pl.when(t > 0)
            def _():
                drain_store(q, sem_s[q])

            transpose_add(s, rows[q], tt[q])
            fire_store(s, tt[q], sem_s[q])
        return 0

    lax.fori_loop(0, SEQ // 4, quad_body, 0)
    for q in range(4):
        drain_store(q, sem_s[q])


def kernel(words, word_table, pos_table):
    wt_t = word_table.T
    tab128 = _detile_tc(wt_t, wt_t)
    tab_lin = tab128.reshape(2 * P2, DIM)

    w = words.astype(jnp.int32)
    g = ((w & (P2 - 1)) << 1) | (w >> 19)
    idx3 = g.T.reshape(SEQ, NW, BBLK).transpose(1, 0, 2)

    out5 = _emb_sc(idx3, tab_lin, pos_table)
    return out5.transpose(2, 4, 0, 1, 3).reshape(BATCH, SEQ, DIM)
